# baseline (speedup 1.0000x reference)
import functools

import jax
import jax.numpy as jnp
from jax import lax
from jax.experimental import pallas as pl
from jax.experimental.pallas import tpu as pltpu
from jax.experimental.pallas import tpu_sc as plsc

_DIM = 768
_EPS = 1e-8
_INV_SQRT_D = 1.0 / (_DIM ** 0.5)

_NBUF = 8
_S_CHUNK = 288

_N_SC_WORKERS = 8


def _sc_gather(idx, table):
    B = idx.shape[0]
    D = table.shape[-1]
    rows_per_w = B // _N_SC_WORKERS
    mesh = plsc.VectorSubcoreMesh(core_axis_name="c", subcore_axis_name="s")

    @functools.partial(
        pl.kernel,
        mesh=mesh,
        out_type=jax.ShapeDtypeStruct((B, D), jnp.float32),
        scratch_types=[
            pltpu.VMEM((rows_per_w,), jnp.int32),
            pltpu.VMEM((rows_per_w, D), jnp.float32),
            pltpu.SemaphoreType.DMA,
        ],
    )
    def k(idx_hbm, table_hbm, out_hbm, idx_v, rows_v, sem):
        wid = lax.axis_index("s") * 2 + lax.axis_index("c")

        @pl.when(wid < _N_SC_WORKERS)
        def _():
            base = wid * rows_per_w
            pltpu.sync_copy(idx_hbm.at[pl.ds(base, rows_per_w)], idx_v)
            pltpu.async_copy(table_hbm.at[idx_v], rows_v, sem).wait()
            pltpu.sync_copy(rows_v, out_hbm.at[pl.ds(base, rows_per_w)])

    return k(idx, table)


def _make_body(B, S, D):
    n_s = S // _S_CHUNK
    n_units = B * n_s

    def body(h_hbm, pe_ref, w_ref, o_hbm, in_buf, out_buf, in_sems, out_sems):
        def in_copy(u, slot):
            b = u // n_s
            s = lax.rem(u, n_s)
            return pltpu.make_async_copy(
                h_hbm.at[b, pl.ds(s * _S_CHUNK, _S_CHUNK), :],
                in_buf.at[slot],
                in_sems.at[slot],
            )

        def out_copy(u, slot):
            b = u // n_s
            s = lax.rem(u, n_s)
            return pltpu.make_async_copy(
                out_buf.at[slot],
                o_hbm.at[b, pl.ds(s * _S_CHUNK, _S_CHUNK), :],
                out_sems.at[slot],
            )

        for i in range(_NBUF):
            in_copy(i, i).start()

        def step(u, carry):
            slot = lax.rem(u, _NBUF)
            in_copy(u, slot).wait()

            @pl.when(u >= _NBUF)
            def _():
                out_copy(u - _NBUF, slot).wait()

            b = u // n_s
            pe_row = pe_ref[b, :]
            x = in_buf[slot] + pe_row[None, :]
            ssq = jnp.sum(x * x, axis=-1, keepdims=True)
            recip = 1.0 / (jnp.sqrt(ssq) * _INV_SQRT_D + _EPS)
            out_buf[slot] = x * (recip * w_ref[...])

            out_copy(u, slot).start()

            @pl.when(u + _NBUF < n_units)
            def _():
                in_copy(u + _NBUF, slot).start()

            return carry

        lax.fori_loop(0, n_units, step, 0)

        for i in range(_NBUF):
            u = n_units - _NBUF + i
            out_copy(u, u % _NBUF).wait()

    return body


def kernel(hidden_state, index, pos_embed, weight):
    B, S, D = hidden_state.shape
    idx = index.astype(jnp.int32)
    table = pos_embed.reshape(pos_embed.shape[0], D)
    pe = _sc_gather(idx, table)
    w2d = weight.reshape(1, D)

    return pl.pallas_call(
        _make_body(B, S, D),
        grid=(1,),
        in_specs=[
            pl.BlockSpec(memory_space=pl.ANY),
            pl.BlockSpec((B, D), lambda i: (0, 0)),
            pl.BlockSpec((1, D), lambda i: (0, 0)),
        ],
        out_specs=pl.BlockSpec(memory_space=pl.ANY),
        out_shape=jax.ShapeDtypeStruct((B, S, D), jnp.float32),
        scratch_shapes=[
            pltpu.VMEM((_NBUF, _S_CHUNK, D), jnp.float32),
            pltpu.VMEM((_NBUF, _S_CHUNK, D), jnp.float32),
            pltpu.SemaphoreType.DMA((_NBUF,)),
            pltpu.SemaphoreType.DMA((_NBUF,)),
        ],
    )(hidden_state, pe, w2d)


# speedup vs baseline: 1.2677x; 1.2677x over previous
import jax
import jax.numpy as jnp
from jax import lax
from jax.experimental import pallas as pl
from jax.experimental.pallas import tpu as pltpu

_DIM = 768
_EPS = 1e-8
_INV_SQRT_D = 1.0 / (_DIM ** 0.5)

_NBUF = 16
_S_CHUNK = 144


def _make_body(B, S, D):
    n_s = S // _S_CHUNK
    n_units = B * n_s

    def body(idx_ref, h_hbm, pe_ref, w_ref, o_hbm,
             in_buf, out_buf, in_sems, out_sems):
        def in_copy(u, slot):
            b = u // n_s
            s = lax.rem(u, n_s)
            return pltpu.make_async_copy(
                h_hbm.at[b, pl.ds(s * _S_CHUNK, _S_CHUNK), :],
                in_buf.at[slot],
                in_sems.at[slot],
            )

        def out_copy(u, slot):
            b = u // n_s
            s = lax.rem(u, n_s)
            return pltpu.make_async_copy(
                out_buf.at[slot],
                o_hbm.at[b, pl.ds(s * _S_CHUNK, _S_CHUNK), :],
                out_sems.at[slot],
            )

        for i in range(_NBUF):
            in_copy(i, i).start()

        def step(u, carry):
            slot = lax.rem(u, _NBUF)
            in_copy(u, slot).wait()

            @pl.when(u >= _NBUF)
            def _():
                out_copy(u - _NBUF, slot).wait()

            b = u // n_s
            pe_row = pe_ref[idx_ref[b], 0, :]
            x = in_buf[slot] + pe_row[None, :]
            ssq = jnp.sum(x * x, axis=-1, keepdims=True)
            recip = 1.0 / (jnp.sqrt(ssq) * _INV_SQRT_D + _EPS)
            out_buf[slot] = x * (recip * w_ref[...])

            out_copy(u, slot).start()

            @pl.when(u + _NBUF < n_units)
            def _():
                in_copy(u + _NBUF, slot).start()

            return carry

        lax.fori_loop(0, n_units, step, 0)

        for i in range(_NBUF):
            u = n_units - _NBUF + i
            out_copy(u, u % _NBUF).wait()

    return body


def kernel(hidden_state, index, pos_embed, weight):
    B, S, D = hidden_state.shape
    idx = index.astype(jnp.int32)
    w2d = weight.reshape(1, D)

    grid_spec = pltpu.PrefetchScalarGridSpec(
        num_scalar_prefetch=1,
        grid=(1,),
        in_specs=[
            pl.BlockSpec(memory_space=pl.ANY),
            pl.BlockSpec((pos_embed.shape[0], 1, D), lambda i, idx_ref: (0, 0, 0)),
            pl.BlockSpec((1, D), lambda i, idx_ref: (0, 0)),
        ],
        out_specs=pl.BlockSpec(memory_space=pl.ANY),
        scratch_shapes=[
            pltpu.VMEM((_NBUF, _S_CHUNK, D), jnp.float32),
            pltpu.VMEM((_NBUF, _S_CHUNK, D), jnp.float32),
            pltpu.SemaphoreType.DMA((_NBUF,)),
            pltpu.SemaphoreType.DMA((_NBUF,)),
        ],
    )
    return pl.pallas_call(
        _make_body(B, S, D),
        grid_spec=grid_spec,
        out_shape=jax.ShapeDtypeStruct((B, S, D), jnp.float32),
    )(idx, hidden_state, pos_embed, w2d)
